# Initial kernel scaffold; baseline (speedup 1.0000x reference)
import jax
import jax.numpy as jnp
from jax.experimental import pallas as pl


def kernel(inputs, embeddings):
    raise NotImplementedError("write your pallas kernel here")


# speedup vs baseline: 1.4985x; 1.4985x over previous
import jax
import jax.numpy as jnp
from jax.experimental import pallas as pl

_ROWS_PER_BLOCK = 512


def _add_kernel(x_ref, e_ref, o_ref):
    o_ref[...] = x_ref[...] + e_ref[...]


def kernel(inputs, embeddings):
    B, S, D = inputs.shape
    bs = _ROWS_PER_BLOCK
    sblk = S // bs
    x = inputs.reshape(B * S, D)
    out = pl.pallas_call(
        _add_kernel,
        grid=(sblk, B),
        in_specs=[
            pl.BlockSpec((bs, D), lambda s, b: (b * sblk + s, 0)),
            pl.BlockSpec((bs, D), lambda s, b: (s, 0)),
        ],
        out_specs=pl.BlockSpec((bs, D), lambda s, b: (b * sblk + s, 0)),
        out_shape=jax.ShapeDtypeStruct((B * S, D), inputs.dtype),
    )(x, embeddings)
    return out.reshape(B, S, D)


# speedup vs baseline: 1.6669x; 1.1124x over previous
import jax
import jax.numpy as jnp
from jax.experimental import pallas as pl

_ROWS_PER_BLOCK = 1024


def _add_kernel(x_ref, e_ref, o_ref):
    o_ref[...] = x_ref[...] + e_ref[...]


def kernel(inputs, embeddings):
    B, S, D = inputs.shape
    bs = _ROWS_PER_BLOCK
    sblk = S // bs
    x = inputs.reshape(B * S, D)
    out = pl.pallas_call(
        _add_kernel,
        grid=(sblk, B),
        in_specs=[
            pl.BlockSpec((bs, D), lambda s, b: (b * sblk + s, 0)),
            pl.BlockSpec((bs, D), lambda s, b: (s, 0)),
        ],
        out_specs=pl.BlockSpec((bs, D), lambda s, b: (b * sblk + s, 0)),
        out_shape=jax.ShapeDtypeStruct((B * S, D), inputs.dtype),
    )(x, embeddings)
    return out.reshape(B, S, D)


# speedup vs baseline: 1.7422x; 1.0451x over previous
import jax
import jax.numpy as jnp
from jax.experimental import pallas as pl

_ROWS_PER_BLOCK = 2048


def _add_kernel(x_ref, e_ref, o_ref):
    o_ref[...] = x_ref[...] + e_ref[...]


def kernel(inputs, embeddings):
    B, S, D = inputs.shape
    bs = _ROWS_PER_BLOCK
    sblk = S // bs
    x = inputs.reshape(B * S, D)
    out = pl.pallas_call(
        _add_kernel,
        grid=(sblk, B),
        in_specs=[
            pl.BlockSpec((bs, D), lambda s, b: (b * sblk + s, 0)),
            pl.BlockSpec((bs, D), lambda s, b: (s, 0)),
        ],
        out_specs=pl.BlockSpec((bs, D), lambda s, b: (b * sblk + s, 0)),
        out_shape=jax.ShapeDtypeStruct((B * S, D), inputs.dtype),
    )(x, embeddings)
    return out.reshape(B, S, D)
